# baseline (speedup 1.0000x reference)
import functools

import jax
import jax.numpy as jnp
from jax import lax
from jax.experimental import pallas as pl
from jax.experimental.pallas import tpu as pltpu
from jax.experimental.pallas import tpu_sc as plsc

B = 4096
NH = 2
BH = B // NH
D_IN = 3 * 32 * 32
PRE = 2048
PART = 2048
NPART = 8
PSZ = PART // NPART
NC = 1000
NCP = 1024
ND = 2
EPS = 1e-5
BT = 256
NBH = BH // BT
NT3 = NBH + NPART - 1
PADB = NT3 * BT
NW = 32
CHUNK = BH // NW
G16 = CHUNK // 16

_DNT = (((1,), (1,)), ((), ()))


def _k1(x_ref, w_ref, b_ref, g_ref, bb_ref, f_ref):
    f = lax.dot_general(x_ref[...], w_ref[...], _DNT,
                        preferred_element_type=jnp.float32)
    f = f + b_ref[...]
    mu = jnp.mean(f, axis=1, keepdims=True)
    var = jnp.mean((f - mu) ** 2, axis=1, keepdims=True)
    f = (f - mu) / jnp.sqrt(var + EPS) * g_ref[...] + bb_ref[...]
    f_ref[...] = jnp.maximum(f, 0.0)


def _k2(f_ref, wd_ref, s_ref, t_ref, wdf_ref, bdf_ref, wps_ref, bps_ref,
        gum_ref, dom_ref, idx_ref, probs_ref, rank_ref, counts_ref, cnt_ref):
    i = pl.program_id(0)
    d = lax.dot_general(f_ref[...], wd_ref[...], _DNT,
                        preferred_element_type=jnp.float32)
    d = jnp.maximum(d * s_ref[...] + t_ref[...], 0.0)
    dom_ref[...] = lax.dot_general(d, wdf_ref[...], _DNT,
                                   preferred_element_type=jnp.float32) + bdf_ref[...]
    z = lax.dot_general(d, wps_ref[...], _DNT,
                        preferred_element_type=jnp.float32) + bps_ref[...]
    z = z + gum_ref[...]
    idx = jnp.argmax(z, axis=1).astype(jnp.int32)
    idx_ref[...] = idx[:, None]
    onehot = (lax.broadcasted_iota(jnp.int32, (BT, NPART), 1)
              == idx[:, None]).astype(jnp.float32)
    probs_ref[...] = onehot

    @pl.when(i == 0)
    def _init():
        cnt_ref[...] = jnp.zeros((1, NPART), jnp.float32)

    tril = (lax.broadcasted_iota(jnp.int32, (BT, BT), 0)
            > lax.broadcasted_iota(jnp.int32, (BT, BT), 1)).astype(jnp.float32)
    csum_ex = jnp.dot(tril, onehot, preferred_element_type=jnp.float32)
    rank = jnp.sum((csum_ex + cnt_ref[...]) * onehot, axis=1)
    rank_ref[...] = rank.astype(jnp.int32)[:, None]
    cnt_new = cnt_ref[...] + jnp.sum(onehot, axis=0, keepdims=True)
    cnt_ref[...] = cnt_new
    counts_ref[...] = cnt_new


def _kdst(probs_ref, rank_ref, off_ref, dst_ref):
    o = jnp.sum(probs_ref[...] * off_ref[...], axis=1)
    dst_ref[...] = o.astype(jnp.int32)[:, None] + rank_ref[...]


def _k3(texp_ref, fs_ref, w1_ref, b1_ref, w2_ref, b2_ref, out_ref):
    h = lax.dot_general(fs_ref[...], w1_ref[0], _DNT,
                        preferred_element_type=jnp.float32)
    h = jnp.maximum(h + b1_ref[0], 0.0)
    out = lax.dot_general(h, w2_ref[0], _DNT,
                          preferred_element_type=jnp.float32)
    out_ref[:, :NC] = out + b2_ref[0]


def _sc_scatter(f_hbm, dst_hbm, fs_hbm, dst_v, rows_v, sem):
    c = lax.axis_index("c")
    s = lax.axis_index("s")
    wid = s * 2 + c
    base = wid * CHUNK
    pltpu.sync_copy(dst_hbm.at[pl.ds(base, CHUNK)], dst_v)
    for g in range(G16):
        dst16 = dst_v[pl.ds(g * 16, 16)]
        pltpu.sync_copy(f_hbm.at[pl.ds(base + g * 16, 16)], rows_v)
        pltpu.async_copy(rows_v, fs_hbm.at[dst16], sem).wait()


def _sc_gather(os_hbm, dst_hbm, out_hbm, dst_v, rows_v, sem):
    c = lax.axis_index("c")
    s = lax.axis_index("s")
    wid = s * 2 + c
    base = wid * CHUNK
    pltpu.sync_copy(dst_hbm.at[pl.ds(base, CHUNK)], dst_v)
    for g in range(G16):
        dst16 = dst_v[pl.ds(g * 16, 16)]
        pltpu.async_copy(os_hbm.at[dst16], rows_v, sem).wait()
        pltpu.sync_copy(rows_v, out_hbm.at[pl.ds(base + g * 16, 16)])


_sc_scatter_call = functools.partial(
    pl.kernel,
    out_type=jax.ShapeDtypeStruct((PADB, PRE), jnp.float32),
    mesh=plsc.VectorSubcoreMesh(core_axis_name="c", subcore_axis_name="s"),
    scratch_types=[
        pltpu.VMEM((CHUNK,), jnp.int32),
        pltpu.VMEM((16, PRE), jnp.float32),
        pltpu.SemaphoreType.DMA,
    ],
)(_sc_scatter)

_sc_gather_call = functools.partial(
    pl.kernel,
    out_type=jax.ShapeDtypeStruct((BH, NCP), jnp.float32),
    mesh=plsc.VectorSubcoreMesh(core_axis_name="c", subcore_axis_name="s"),
    scratch_types=[
        pltpu.VMEM((CHUNK,), jnp.int32),
        pltpu.VMEM((16, NCP), jnp.float32),
        pltpu.SemaphoreType.DMA,
    ],
)(_sc_gather)


def _stream(xh, gum_h, W_pre, b_pre1, ln_g1, ln_b1, W_d, s, t, W_df, b_df1,
            W_ps, b_ps1, Wp1, bp1r, Wp2, bp2r):
    f = pl.pallas_call(
        _k1,
        grid=(NBH,),
        in_specs=[
            pl.BlockSpec((BT, D_IN), lambda i: (i, 0)),
            pl.BlockSpec((PRE, D_IN), lambda i: (0, 0)),
            pl.BlockSpec((1, PRE), lambda i: (0, 0)),
            pl.BlockSpec((1, PRE), lambda i: (0, 0)),
            pl.BlockSpec((1, PRE), lambda i: (0, 0)),
        ],
        out_specs=pl.BlockSpec((BT, PRE), lambda i: (i, 0)),
        out_shape=jax.ShapeDtypeStruct((BH, PRE), jnp.float32),
    )(xh, W_pre, b_pre1, ln_g1, ln_b1)

    dom, idx2, probs, rank2, counts = pl.pallas_call(
        _k2,
        grid=(NBH,),
        in_specs=[
            pl.BlockSpec((BT, PRE), lambda i: (i, 0)),
            pl.BlockSpec((PART, PRE), lambda i: (0, 0)),
            pl.BlockSpec((1, PART), lambda i: (0, 0)),
            pl.BlockSpec((1, PART), lambda i: (0, 0)),
            pl.BlockSpec((ND, PART), lambda i: (0, 0)),
            pl.BlockSpec((1, ND), lambda i: (0, 0)),
            pl.BlockSpec((NPART, PART), lambda i: (0, 0)),
            pl.BlockSpec((1, NPART), lambda i: (0, 0)),
            pl.BlockSpec((BT, NPART), lambda i: (i, 0)),
        ],
        out_specs=[
            pl.BlockSpec((BT, ND), lambda i: (i, 0)),
            pl.BlockSpec((BT, 1), lambda i: (i, 0)),
            pl.BlockSpec((BT, NPART), lambda i: (i, 0)),
            pl.BlockSpec((BT, 1), lambda i: (i, 0)),
            pl.BlockSpec((1, NPART), lambda i: (0, 0)),
        ],
        out_shape=[
            jax.ShapeDtypeStruct((BH, ND), jnp.float32),
            jax.ShapeDtypeStruct((BH, 1), jnp.int32),
            jax.ShapeDtypeStruct((BH, NPART), jnp.float32),
            jax.ShapeDtypeStruct((BH, 1), jnp.int32),
            jax.ShapeDtypeStruct((1, NPART), jnp.float32),
        ],
        scratch_shapes=[pltpu.VMEM((1, NPART), jnp.float32)],
    )(f, W_d, s, t, W_df, b_df1, W_ps, b_ps1, gum_h)

    counts_i = counts.reshape(NPART).astype(jnp.int32)
    padded = ((counts_i + BT - 1) // BT) * BT
    csum = jnp.cumsum(padded)
    off = jnp.concatenate([jnp.zeros((1,), jnp.int32), csum[:-1]])
    start_tile = off // BT
    tt = jnp.arange(NT3, dtype=jnp.int32)
    texp = jnp.sum((tt[:, None] >= start_tile[None, :]).astype(jnp.int32),
                   axis=1) - 1
    texp = jnp.clip(texp, 0, NPART - 1)

    dst2 = pl.pallas_call(
        _kdst,
        grid=(NBH,),
        in_specs=[
            pl.BlockSpec((BT, NPART), lambda i: (i, 0)),
            pl.BlockSpec((BT, 1), lambda i: (i, 0)),
            pl.BlockSpec((1, NPART), lambda i: (0, 0)),
        ],
        out_specs=pl.BlockSpec((BT, 1), lambda i: (i, 0)),
        out_shape=jax.ShapeDtypeStruct((BH, 1), jnp.int32),
    )(probs, rank2, off.astype(jnp.float32).reshape(1, NPART))
    dst_flat = dst2.reshape(BH)

    fs = _sc_scatter_call(f, dst_flat)

    out_sorted = pl.pallas_call(
        _k3,
        grid_spec=pltpu.PrefetchScalarGridSpec(
            num_scalar_prefetch=1,
            grid=(NT3,),
            in_specs=[
                pl.BlockSpec((BT, PRE), lambda i, tx: (i, 0)),
                pl.BlockSpec((1, PSZ, PRE), lambda i, tx: (tx[i], 0, 0)),
                pl.BlockSpec((1, 1, PSZ), lambda i, tx: (tx[i], 0, 0)),
                pl.BlockSpec((1, NC, PSZ), lambda i, tx: (tx[i], 0, 0)),
                pl.BlockSpec((1, 1, NC), lambda i, tx: (tx[i], 0, 0)),
            ],
            out_specs=pl.BlockSpec((BT, NCP), lambda i, tx: (i, 0)),
        ),
        out_shape=jax.ShapeDtypeStruct((PADB, NCP), jnp.float32),
    )(texp, fs, Wp1, bp1r, Wp2, bp2r)

    class_pad = _sc_gather_call(out_sorted, dst_flat)
    return class_pad, dom, idx2, probs


def kernel(input_data, W_pre, b_pre, ln_g, ln_b, W_d, b_d, bnd_g, bnd_b,
           bnd_mean, bnd_var, W_df, b_df, W_ps, b_ps, Wp1, bp1, Wp2, bp2,
           gumbel):
    s = (bnd_g * lax.rsqrt(bnd_var + EPS)).reshape(1, PART)
    t = (bnd_b - bnd_mean * s.reshape(PART)).reshape(1, PART)
    wargs = (W_pre, b_pre.reshape(1, PRE), ln_g.reshape(1, PRE),
             ln_b.reshape(1, PRE), W_d, s, t, W_df, b_df.reshape(1, ND),
             W_ps, b_ps.reshape(1, NPART), Wp1, bp1.reshape(NPART, 1, PSZ),
             Wp2, bp2.reshape(NPART, 1, NC))

    res = []
    for h in range(NH):
        xh = input_data[h * BH:(h + 1) * BH].reshape(BH, D_IN)
        gum_h = gumbel[h * BH:(h + 1) * BH]
        res.append(_stream(xh, gum_h, *wargs))

    class_out = jnp.concatenate([r[0][:, :NC] for r in res], axis=0)
    dom = jnp.concatenate([r[1] for r in res], axis=0)
    idx = jnp.concatenate([r[2].reshape(BH) for r in res], axis=0)
    probs = jnp.concatenate([r[3] for r in res], axis=0)
    return (class_out, dom, idx, probs)


# speedup vs baseline: 1.2088x; 1.2088x over previous
import functools

import jax
import jax.numpy as jnp
from jax import lax
from jax.experimental import pallas as pl
from jax.experimental.pallas import tpu as pltpu
from jax.experimental.pallas import tpu_sc as plsc

B = 4096
D_IN = 3 * 32 * 32
PRE = 2048
PART = 2048
NPART = 8
PSZ = PART // NPART
NC = 1000
NCP = 1024
ND = 2
EPS = 1e-5
BT = 256
NBT = B // BT
NT3 = NBT + NPART - 1
PADB = NT3 * BT
NW = 32
CHUNK = B // NW
G16 = CHUNK // 16

_DNT = (((1,), (1,)), ((), ()))


def _k1(x_ref, w_ref, b_ref, g_ref, bb_ref, f_ref):
    f = lax.dot_general(x_ref[...], w_ref[...], _DNT,
                        preferred_element_type=jnp.float32)
    f = f + b_ref[...]
    mu = jnp.mean(f, axis=1, keepdims=True)
    var = jnp.mean((f - mu) ** 2, axis=1, keepdims=True)
    f = (f - mu) / jnp.sqrt(var + EPS) * g_ref[...] + bb_ref[...]
    f_ref[...] = jnp.maximum(f, 0.0)


def _k2(f_ref, wd_ref, s_ref, t_ref, wdf_ref, bdf_ref, wps_ref, bps_ref,
        gum_ref, dom_ref, idx_ref, probs_ref, rank_ref, counts_ref, cnt_ref):
    i = pl.program_id(0)
    d = lax.dot_general(f_ref[...], wd_ref[...], _DNT,
                        preferred_element_type=jnp.float32)
    d = jnp.maximum(d * s_ref[...] + t_ref[...], 0.0)
    dom_ref[...] = lax.dot_general(d, wdf_ref[...], _DNT,
                                   preferred_element_type=jnp.float32) + bdf_ref[...]
    z = lax.dot_general(d, wps_ref[...], _DNT,
                        preferred_element_type=jnp.float32) + bps_ref[...]
    z = z + gum_ref[...]
    idx = jnp.argmax(z, axis=1).astype(jnp.int32)
    idx_ref[...] = idx[:, None]
    onehot = (lax.broadcasted_iota(jnp.int32, (BT, NPART), 1)
              == idx[:, None]).astype(jnp.float32)
    probs_ref[...] = onehot

    @pl.when(i == 0)
    def _init():
        cnt_ref[...] = jnp.zeros((1, NPART), jnp.float32)

    tril = (lax.broadcasted_iota(jnp.int32, (BT, BT), 0)
            > lax.broadcasted_iota(jnp.int32, (BT, BT), 1)).astype(jnp.float32)
    csum_ex = jnp.dot(tril, onehot, preferred_element_type=jnp.float32)
    rank = jnp.sum((csum_ex + cnt_ref[...]) * onehot, axis=1)
    rank_ref[...] = rank.astype(jnp.int32)[:, None]
    cnt_new = cnt_ref[...] + jnp.sum(onehot, axis=0, keepdims=True)
    cnt_ref[...] = cnt_new
    counts_ref[...] = cnt_new


def _k3(texp_ref, fs_ref, w1_ref, b1_ref, w2_ref, b2_ref, out_ref):
    h = lax.dot_general(fs_ref[...], w1_ref[0], _DNT,
                        preferred_element_type=jnp.float32)
    h = jnp.maximum(h + b1_ref[0], 0.0)
    out = lax.dot_general(h, w2_ref[0], _DNT,
                          preferred_element_type=jnp.float32)
    out_ref[:, :NC] = out + b2_ref[0]


def _kdst(probs_ref, rank_ref, off_ref, dst_ref):
    o = jnp.sum(probs_ref[...] * off_ref[...], axis=1)
    dst_ref[...] = o.astype(jnp.int32)[:, None] + rank_ref[...]


def _sc_scatter(f_hbm, dst_hbm, fs_hbm, dst_v, rows_a, rows_b, sem_ra,
                sem_rb, sem_wa, sem_wb):
    c = lax.axis_index("c")
    s = lax.axis_index("s")
    wid = s * 2 + c
    base = wid * CHUNK
    pltpu.sync_copy(dst_hbm.at[pl.ds(base, CHUNK)], dst_v)
    bufs = (rows_a, rows_b)
    rsem = (sem_ra, sem_rb)
    wsem = (sem_wa, sem_wb)
    reads = [None] * G16
    writes = [None] * G16
    reads[0] = pltpu.async_copy(f_hbm.at[pl.ds(base, 16)], bufs[0], rsem[0])
    for g in range(G16):
        cur = g % 2
        reads[g].wait()
        if g >= 1:
            writes[g - 1].wait()
        if g + 1 < G16:
            reads[g + 1] = pltpu.async_copy(
                f_hbm.at[pl.ds(base + (g + 1) * 16, 16)],
                bufs[1 - cur], rsem[1 - cur])
        dst16 = dst_v[pl.ds(g * 16, 16)]
        writes[g] = pltpu.async_copy(bufs[cur], fs_hbm.at[dst16], wsem[cur])
    writes[G16 - 1].wait()


def _sc_gather(os_hbm, dst_hbm, out_hbm, dst_v, rows_a, rows_b, sem_ra,
               sem_rb, sem_wa, sem_wb):
    c = lax.axis_index("c")
    s = lax.axis_index("s")
    wid = s * 2 + c
    base = wid * CHUNK
    pltpu.sync_copy(dst_hbm.at[pl.ds(base, CHUNK)], dst_v)
    bufs = (rows_a, rows_b)
    rsem = (sem_ra, sem_rb)
    wsem = (sem_wa, sem_wb)
    reads = [None] * G16
    writes = [None] * G16
    reads[0] = pltpu.async_copy(os_hbm.at[dst_v[pl.ds(0, 16)]], bufs[0],
                                rsem[0])
    for g in range(G16):
        cur = g % 2
        reads[g].wait()
        if g >= 1:
            writes[g - 1].wait()
        if g + 1 < G16:
            reads[g + 1] = pltpu.async_copy(
                os_hbm.at[dst_v[pl.ds((g + 1) * 16, 16)]],
                bufs[1 - cur], rsem[1 - cur])
        writes[g] = pltpu.async_copy(
            bufs[cur], out_hbm.at[pl.ds(base + g * 16, 16)], wsem[cur])
    writes[G16 - 1].wait()


def kernel(input_data, W_pre, b_pre, ln_g, ln_b, W_d, b_d, bnd_g, bnd_b,
           bnd_mean, bnd_var, W_df, b_df, W_ps, b_ps, Wp1, bp1, Wp2, bp2,
           gumbel):
    x = input_data.reshape(B, D_IN)
    s = (bnd_g * lax.rsqrt(bnd_var + EPS)).reshape(1, PART)
    t = (bnd_b - bnd_mean * s.reshape(PART)).reshape(1, PART)

    f = pl.pallas_call(
        _k1,
        grid=(NBT,),
        in_specs=[
            pl.BlockSpec((BT, D_IN), lambda i: (i, 0)),
            pl.BlockSpec((PRE, D_IN), lambda i: (0, 0)),
            pl.BlockSpec((1, PRE), lambda i: (0, 0)),
            pl.BlockSpec((1, PRE), lambda i: (0, 0)),
            pl.BlockSpec((1, PRE), lambda i: (0, 0)),
        ],
        out_specs=pl.BlockSpec((BT, PRE), lambda i: (i, 0)),
        out_shape=jax.ShapeDtypeStruct((B, PRE), jnp.float32),
    )(x, W_pre, b_pre.reshape(1, PRE), ln_g.reshape(1, PRE),
      ln_b.reshape(1, PRE))

    dom, idx2, probs, rank2, counts = pl.pallas_call(
        _k2,
        grid=(NBT,),
        in_specs=[
            pl.BlockSpec((BT, PRE), lambda i: (i, 0)),
            pl.BlockSpec((PART, PRE), lambda i: (0, 0)),
            pl.BlockSpec((1, PART), lambda i: (0, 0)),
            pl.BlockSpec((1, PART), lambda i: (0, 0)),
            pl.BlockSpec((ND, PART), lambda i: (0, 0)),
            pl.BlockSpec((1, ND), lambda i: (0, 0)),
            pl.BlockSpec((NPART, PART), lambda i: (0, 0)),
            pl.BlockSpec((1, NPART), lambda i: (0, 0)),
            pl.BlockSpec((BT, NPART), lambda i: (i, 0)),
        ],
        out_specs=[
            pl.BlockSpec((BT, ND), lambda i: (i, 0)),
            pl.BlockSpec((BT, 1), lambda i: (i, 0)),
            pl.BlockSpec((BT, NPART), lambda i: (i, 0)),
            pl.BlockSpec((BT, 1), lambda i: (i, 0)),
            pl.BlockSpec((1, NPART), lambda i: (0, 0)),
        ],
        out_shape=[
            jax.ShapeDtypeStruct((B, ND), jnp.float32),
            jax.ShapeDtypeStruct((B, 1), jnp.int32),
            jax.ShapeDtypeStruct((B, NPART), jnp.float32),
            jax.ShapeDtypeStruct((B, 1), jnp.int32),
            jax.ShapeDtypeStruct((1, NPART), jnp.float32),
        ],
        scratch_shapes=[pltpu.VMEM((1, NPART), jnp.float32)],
    )(f, W_d, s, t, W_df, b_df.reshape(1, ND), W_ps, b_ps.reshape(1, NPART),
      gumbel)

    counts_i = counts.reshape(NPART).astype(jnp.int32)
    padded = ((counts_i + BT - 1) // BT) * BT
    csum = jnp.cumsum(padded)
    off = jnp.concatenate([jnp.zeros((1,), jnp.int32), csum[:-1]])
    start_tile = off // BT
    tt = jnp.arange(NT3, dtype=jnp.int32)
    texp = jnp.sum((tt[:, None] >= start_tile[None, :]).astype(jnp.int32),
                   axis=1) - 1
    texp = jnp.clip(texp, 0, NPART - 1)

    dst2 = pl.pallas_call(
        _kdst,
        grid=(NBT,),
        in_specs=[
            pl.BlockSpec((BT, NPART), lambda i: (i, 0)),
            pl.BlockSpec((BT, 1), lambda i: (i, 0)),
            pl.BlockSpec((1, NPART), lambda i: (0, 0)),
        ],
        out_specs=pl.BlockSpec((BT, 1), lambda i: (i, 0)),
        out_shape=jax.ShapeDtypeStruct((B, 1), jnp.int32),
    )(probs, rank2, off.astype(jnp.float32).reshape(1, NPART))
    dst_flat = dst2.reshape(B)

    sc_scatter = functools.partial(
        pl.kernel,
        out_type=jax.ShapeDtypeStruct((PADB, PRE), jnp.float32),
        mesh=plsc.VectorSubcoreMesh(core_axis_name="c", subcore_axis_name="s"),
        scratch_types=[
            pltpu.VMEM((CHUNK,), jnp.int32),
            pltpu.VMEM((16, PRE), jnp.float32),
            pltpu.VMEM((16, PRE), jnp.float32),
            pltpu.SemaphoreType.DMA,
            pltpu.SemaphoreType.DMA,
            pltpu.SemaphoreType.DMA,
            pltpu.SemaphoreType.DMA,
        ],
    )(_sc_scatter)
    fs = sc_scatter(f, dst_flat)

    out_sorted = pl.pallas_call(
        _k3,
        grid_spec=pltpu.PrefetchScalarGridSpec(
            num_scalar_prefetch=1,
            grid=(NT3,),
            in_specs=[
                pl.BlockSpec((BT, PRE), lambda i, tx: (i, 0)),
                pl.BlockSpec((1, PSZ, PRE), lambda i, tx: (tx[i], 0, 0)),
                pl.BlockSpec((1, 1, PSZ), lambda i, tx: (tx[i], 0, 0)),
                pl.BlockSpec((1, NC, PSZ), lambda i, tx: (tx[i], 0, 0)),
                pl.BlockSpec((1, 1, NC), lambda i, tx: (tx[i], 0, 0)),
            ],
            out_specs=pl.BlockSpec((BT, NCP), lambda i, tx: (i, 0)),
        ),
        out_shape=jax.ShapeDtypeStruct((PADB, NCP), jnp.float32),
    )(texp, fs, Wp1, bp1.reshape(NPART, 1, PSZ), Wp2,
      bp2.reshape(NPART, 1, NC))

    sc_gather = functools.partial(
        pl.kernel,
        out_type=jax.ShapeDtypeStruct((B, NCP), jnp.float32),
        mesh=plsc.VectorSubcoreMesh(core_axis_name="c", subcore_axis_name="s"),
        scratch_types=[
            pltpu.VMEM((CHUNK,), jnp.int32),
            pltpu.VMEM((16, NCP), jnp.float32),
            pltpu.VMEM((16, NCP), jnp.float32),
            pltpu.SemaphoreType.DMA,
            pltpu.SemaphoreType.DMA,
            pltpu.SemaphoreType.DMA,
            pltpu.SemaphoreType.DMA,
        ],
    )(_sc_gather)
    class_pad = sc_gather(out_sorted, dst_flat)

    return (class_pad[:, :NC], dom, idx2.reshape(B), probs)
